# baseline (speedup 1.0000x reference)
import functools

import jax
import jax.numpy as jnp
from jax import lax
from jax.experimental import pallas as pl
from jax.experimental.pallas import tpu as pltpu
from jax.experimental.pallas import tpu_sc as plsc

_NC = 2
_NS = 16
_NW = _NC * _NS

_DW = 32
_GROUP = 128
_G_CHUNK = 8


def _gather_body(n_super, g_per_worker, idx_hbm, w_hbm, out_hbm,
                 idx_v, rows_v, sem):
    wid = lax.axis_index("s") * _NC + lax.axis_index("c")

    def super_body(i, carry):
        gbase = wid * g_per_worker + i * _G_CHUNK
        pltpu.sync_copy(idx_hbm.at[pl.ds(gbase, _G_CHUNK)], idx_v)
        copies = [
            pltpu.async_copy(w_hbm.at[idx_v.at[g]], rows_v.at[g], sem)
            for g in range(_G_CHUNK)
        ]
        for cp in copies:
            cp.wait()
        pltpu.sync_copy(rows_v, out_hbm.at[pl.ds(gbase, _G_CHUNK)])
        return carry

    lax.fori_loop(0, n_super, super_body, 0)


def kernel(input_ids, w):
    batch, seq = input_ids.shape
    n_vocab, d_emb = w.shape
    n_idx = batch * seq
    assert d_emb == 2 * _DW
    assert n_idx % (_NW * _GROUP * _G_CHUNK) == 0

    g_total = n_idx // _GROUP
    g_per_worker = g_total // _NW
    n_super = g_per_worker // _G_CHUNK

    idx2 = input_ids.reshape(g_total, _GROUP)
    w32 = lax.bitcast_convert_type(w.reshape(n_vocab, _DW, 2), jnp.int32)

    mesh = plsc.VectorSubcoreMesh(core_axis_name="c", subcore_axis_name="s")
    run = pl.kernel(
        functools.partial(_gather_body, n_super, g_per_worker),
        out_type=jax.ShapeDtypeStruct((g_total, _GROUP, _DW), jnp.int32),
        mesh=mesh,
        scratch_types=[
            pltpu.VMEM((_G_CHUNK, _GROUP), jnp.int32),
            pltpu.VMEM((_G_CHUNK, _GROUP, _DW), jnp.int32),
            pltpu.SemaphoreType.DMA,
        ],
        compiler_params=pltpu.CompilerParams(use_tc_tiling_on_sc=False),
    )
    out32 = run(idx2, w32)
    out = lax.bitcast_convert_type(out32, jnp.bfloat16)
    return out.reshape(batch, seq, d_emb)


# speedup vs baseline: 1.0077x; 1.0077x over previous
import functools

import jax
import jax.numpy as jnp
from jax import lax
from jax.experimental import pallas as pl
from jax.experimental.pallas import tpu as pltpu
from jax.experimental.pallas import tpu_sc as plsc

_NC = 2
_NS = 16
_NW = _NC * _NS

_DW = 32
_GROUP = 128
_K = 5
_NB = 4
_MAIN_UNROLL = 4


def _gather_body(g_per_worker, idx_hbm, w_hbm, out_hbm, idx_v,
                 rows0, rows1, rows2, rows3,
                 gsem0, gsem1, gsem2, gsem3,
                 ssem0, ssem1, ssem2, ssem3):
    wid = lax.axis_index("s") * _NC + lax.axis_index("c")
    base_g = wid * g_per_worker
    n_rounds = g_per_worker // _K

    rows = [rows0, rows1, rows2, rows3]
    gsem = [gsem0, gsem1, gsem2, gsem3]
    ssem = [ssem0, ssem1, ssem2, ssem3]

    pltpu.sync_copy(idx_hbm.at[pl.ds(base_g, g_per_worker)], idx_v)

    def fire_gathers(r, slot):
        for g in range(_K):
            pltpu.async_copy(w_hbm.at[idx_v.at[r * _K + g]],
                             rows[slot].at[g], gsem[slot])

    def out_slice(r):
        return out_hbm.at[pl.ds(base_g + r * _K, _K)]

    def drain_gathers(r, slot):
        pltpu.make_async_copy(out_slice(r), rows[slot], gsem[slot]).wait()

    def fire_store(r, slot):
        pltpu.async_copy(rows[slot], out_slice(r), ssem[slot])

    def wait_store(r_old, slot):
        pltpu.make_async_copy(rows[slot], out_slice(r_old), ssem[slot]).wait()

    fire_gathers(0, 0)
    fire_gathers(1, 1)

    for r in (0, 1):
        fire_gathers(r + 2, r + 2)
        drain_gathers(r, r)
        fire_store(r, r)

    def main_body(i, carry):
        r0 = 2 + i * _MAIN_UNROLL
        for j in range(_MAIN_UNROLL):
            r = r0 + j
            slot = (2 + j) % _NB
            nxt = (slot + 2) % _NB
            wait_store(r - 2, nxt)
            fire_gathers(r + 2, nxt)
            drain_gathers(r, slot)
            fire_store(r, slot)
        return carry

    n_main = (n_rounds - 4) // _MAIN_UNROLL
    lax.fori_loop(0, n_main, main_body, 0)

    for r in (n_rounds - 2, n_rounds - 1):
        slot = r % _NB
        drain_gathers(r, slot)
        fire_store(r, slot)

    for r in range(n_rounds - _NB, n_rounds):
        wait_store(r, r % _NB)


def kernel(input_ids, w):
    batch, seq = input_ids.shape
    n_vocab, d_emb = w.shape
    n_idx = batch * seq
    assert d_emb == 2 * _DW
    assert n_idx % (_NW * _GROUP) == 0

    g_total = n_idx // _GROUP
    g_per_worker = g_total // _NW
    n_rounds = g_per_worker // _K
    assert g_per_worker % _K == 0
    assert (n_rounds - 4) % _MAIN_UNROLL == 0

    idx2 = input_ids.reshape(g_total, _GROUP)
    w32 = lax.bitcast_convert_type(w.reshape(n_vocab, _DW, 2), jnp.int32)

    mesh = plsc.VectorSubcoreMesh(core_axis_name="c", subcore_axis_name="s")
    run = pl.kernel(
        functools.partial(_gather_body, g_per_worker),
        out_type=jax.ShapeDtypeStruct((g_total, _GROUP, _DW), jnp.int32),
        mesh=mesh,
        scratch_types=(
            [pltpu.VMEM((g_per_worker, _GROUP), jnp.int32)]
            + [pltpu.VMEM((_K, _GROUP, _DW), jnp.int32)] * _NB
            + [pltpu.SemaphoreType.DMA] * (2 * _NB)
        ),
        compiler_params=pltpu.CompilerParams(use_tc_tiling_on_sc=False),
    )
    out32 = run(idx2, w32)
    out = lax.bitcast_convert_type(out32, jnp.bfloat16)
    return out.reshape(batch, seq, d_emb)


# speedup vs baseline: 2.2196x; 2.2027x over previous
import functools

import jax
import jax.numpy as jnp
from jax import lax
from jax.experimental import pallas as pl
from jax.experimental.pallas import tpu as pltpu
from jax.experimental.pallas import tpu_sc as plsc

_NC = 2
_NS = 16
_NW = _NC * _NS

_DW = 32
_GROUP = 128
_K = 5
_NB = 4
_MAIN_UNROLL = 4


def _gather_body(g_per_worker, idx_hbm, w_hbm, out_hbm, idx_v,
                 rows0, rows1, rows2, rows3,
                 gsem0, gsem1, gsem2, gsem3,
                 ssem0, ssem1, ssem2, ssem3):
    wid = lax.axis_index("s") * _NC + lax.axis_index("c")
    base_g = wid * g_per_worker
    n_rounds = g_per_worker // _K

    rows = [rows0, rows1, rows2, rows3]
    gsem = [gsem0, gsem1, gsem2, gsem3]
    ssem = [ssem0, ssem1, ssem2, ssem3]

    pltpu.sync_copy(idx_hbm.at[pl.ds(base_g, g_per_worker)], idx_v)

    def fire_gathers(r, slot):
        for g in range(_K):
            pltpu.async_copy(w_hbm.at[idx_v.at[r * _K + g]],
                             rows[slot].at[g], gsem[slot])

    def out_slice(r):
        return out_hbm.at[pl.ds(base_g + r * _K, _K)]

    def drain_gathers(r, slot):
        pltpu.make_async_copy(out_slice(r), rows[slot], gsem[slot]).wait()

    def fire_store(r, slot):
        pltpu.async_copy(rows[slot], out_slice(r), ssem[slot])

    def wait_store(r_old, slot):
        pltpu.make_async_copy(rows[slot], out_slice(r_old), ssem[slot]).wait()

    fire_gathers(0, 0)
    fire_gathers(1, 1)

    for r in (0, 1):
        fire_gathers(r + 2, r + 2)
        drain_gathers(r, r)
        fire_store(r, r)

    def main_body(i, carry):
        r0 = 2 + i * _MAIN_UNROLL
        for j in range(_MAIN_UNROLL):
            r = r0 + j
            slot = (2 + j) % _NB
            nxt = (slot + 2) % _NB
            wait_store(r - 2, nxt)
            fire_gathers(r + 2, nxt)
            drain_gathers(r, slot)
            fire_store(r, slot)
        return carry

    n_main = (n_rounds - 4) // _MAIN_UNROLL
    lax.fori_loop(0, n_main, main_body, 0)

    for r in (n_rounds - 2, n_rounds - 1):
        slot = r % _NB
        drain_gathers(r, slot)
        fire_store(r, slot)

    for r in range(n_rounds - _NB, n_rounds):
        wait_store(r, r % _NB)


def kernel(input_ids, w):
    batch, seq = input_ids.shape
    n_vocab, d_emb = w.shape
    n_idx = batch * seq
    assert d_emb == 2 * _DW
    assert n_idx % (_NW * _GROUP) == 0

    g_total = n_idx // _GROUP
    g_per_worker = g_total // _NW
    n_rounds = g_per_worker // _K
    assert g_per_worker % _K == 0
    assert (n_rounds - 4) % _MAIN_UNROLL == 0

    idx2 = input_ids.reshape(g_total, _GROUP)

    mesh = plsc.VectorSubcoreMesh(core_axis_name="c", subcore_axis_name="s")
    run = pl.kernel(
        functools.partial(_gather_body, g_per_worker),
        out_type=jax.ShapeDtypeStruct((g_total, _GROUP, d_emb), jnp.bfloat16),
        mesh=mesh,
        scratch_types=(
            [pltpu.VMEM((g_per_worker, _GROUP), jnp.int32)]
            + [pltpu.VMEM((_K, _GROUP, d_emb), jnp.bfloat16)] * _NB
            + [pltpu.SemaphoreType.DMA] * (2 * _NB)
        ),
        compiler_params=pltpu.CompilerParams(use_tc_tiling_on_sc=False),
    )
    out = run(idx2, w)
    return out.reshape(batch, seq, d_emb)
